# Initial kernel scaffold; baseline (speedup 1.0000x reference)
import jax
import jax.numpy as jnp
from jax.experimental import pallas as pl


def kernel(x, edge_index, Ws1, Wn1, b1, Ws2, Wn2, b2, Ws3, Wn3, b3):
    raise NotImplementedError("write your pallas kernel here")


# speedup vs baseline: 2.9822x; 2.9822x over previous
import jax
import jax.numpy as jnp
from jax import lax
from jax.experimental import pallas as pl
from jax.experimental.pallas import tpu as pltpu
from jax.experimental.pallas import tpu_sc as plsc

N_NODES = 10000
FEAT = 128
N_CLS = 47
CP = 48
N_PAD = 10240
N_EDGES = 320000
NUM_SC = 2
NSUB = 16
KCH = 128
GRP = 8
NGRP = 10
E_PAD = NUM_SC * NSUB * NGRP * GRP * KCH
ROWS_PER_TILE = N_PAD // NSUB
BM = 1024


def _sc_aggregate(table, srcr, dstr):
    def body(table_h, src_h, dst_h, out_h, src_v, dst_v, gbuf, acc_sh):
        cid = lax.axis_index("c")
        sid = lax.axis_index("s")
        wid = cid * NSUB + sid

        zv = jnp.zeros((16,), jnp.float32)

        @pl.loop(0, KCH)
        def _(r):
            for c2 in range(FEAT // 16):
                gbuf[r, pl.ds(c2 * 16, 16)] = zv

        base = sid * ROWS_PER_TILE
        for j in range(ROWS_PER_TILE // KCH):
            pltpu.sync_copy(gbuf, acc_sh.at[pl.ds(base + j * KCH, KCH)])
        plsc.subcore_barrier()

        @pl.loop(0, NGRP)
        def _(g):
            pltpu.sync_copy(src_h.at[wid, g], src_v)
            pltpu.sync_copy(dst_h.at[wid, g], dst_v)
            for j in range(GRP):
                pltpu.sync_copy(table_h.at[src_v.at[j]], gbuf)
                pltpu.sync_copy(gbuf, acc_sh.at[dst_v.at[j]], add=True)

        plsc.subcore_barrier()

        for j in range(ROWS_PER_TILE // KCH):
            r0 = base + j * KCH
            pltpu.sync_copy(acc_sh.at[pl.ds(r0, KCH)], out_h.at[cid, pl.ds(r0, KCH)])

    mesh = plsc.VectorSubcoreMesh(core_axis_name="c", subcore_axis_name="s")
    k = pl.kernel(
        body,
        out_type=jax.ShapeDtypeStruct((NUM_SC, N_PAD, FEAT), jnp.float32),
        mesh=mesh,
        scratch_types=[
            pltpu.VMEM((GRP, KCH), jnp.int32),
            pltpu.VMEM((GRP, KCH), jnp.int32),
            pltpu.VMEM((KCH, FEAT), jnp.float32),
            pltpu.VMEM_SHARED((N_PAD, FEAT), jnp.float32),
        ])
    return k(table, srcr, dstr)


def _sc_degree(dstr):
    def body(dst_h, deg_h, dst_v, ones_v, deg_sh):
        cid = lax.axis_index("c")
        sid = lax.axis_index("s")
        wid = cid * NSUB + sid
        zv = jnp.zeros((16,), jnp.float32)

        @pl.loop(0, KCH)
        def _(r):
            ones_v[r, :] = zv

        base = sid * ROWS_PER_TILE
        for j in range(ROWS_PER_TILE // KCH):
            pltpu.sync_copy(ones_v, deg_sh.at[pl.ds(base + j * KCH, KCH)])
        ov = jnp.ones((16,), jnp.float32)

        @pl.loop(0, KCH)
        def _(r):
            ones_v[r, :] = ov

        plsc.subcore_barrier()

        @pl.loop(0, NGRP)
        def _(g):
            pltpu.sync_copy(dst_h.at[wid, g], dst_v)
            for j in range(GRP):
                pltpu.sync_copy(ones_v, deg_sh.at[dst_v.at[j]], add=True)

        plsc.subcore_barrier()
        for j in range(ROWS_PER_TILE // KCH):
            r0 = base + j * KCH
            pltpu.sync_copy(deg_sh.at[pl.ds(r0, KCH)], deg_h.at[cid, pl.ds(r0, KCH)])

    mesh = plsc.VectorSubcoreMesh(core_axis_name="c", subcore_axis_name="s")
    k = pl.kernel(
        body,
        out_type=jax.ShapeDtypeStruct((NUM_SC, N_PAD, 16), jnp.float32),
        mesh=mesh,
        scratch_types=[
            pltpu.VMEM((GRP, KCH), jnp.int32),
            pltpu.VMEM((KCH, 16), jnp.float32),
            pltpu.VMEM_SHARED((N_PAD, 16), jnp.float32),
        ],
        compiler_params=pltpu.CompilerParams(use_tc_tiling_on_sc=False))
    return k(dstr)


def _dense(x, Ws, Wn, b):
    def body(x_ref, ws_ref, wn_ref, b_ref, hs_ref, hn_ref):
        xb = x_ref[...]
        hs_ref[...] = jnp.dot(xb, ws_ref[...], preferred_element_type=jnp.float32) + b_ref[...]
        hn_ref[...] = jnp.dot(xb, wn_ref[...], preferred_element_type=jnp.float32)

    return pl.pallas_call(
        body,
        grid=(N_PAD // BM,),
        in_specs=[
            pl.BlockSpec((BM, FEAT), lambda i: (i, 0)),
            pl.BlockSpec((FEAT, FEAT), lambda i: (0, 0)),
            pl.BlockSpec((FEAT, FEAT), lambda i: (0, 0)),
            pl.BlockSpec((1, FEAT), lambda i: (0, 0)),
        ],
        out_specs=[pl.BlockSpec((BM, FEAT), lambda i: (i, 0))] * 2,
        out_shape=[jax.ShapeDtypeStruct((N_PAD, FEAT), jnp.float32)] * 2,
    )(x, Ws, Wn, b.reshape(1, FEAT))


def _combine_dense(hs_prev, aggp, degp, Ws, Wn, b, emit_h):
    def body(hs_ref, ag_ref, dg_ref, ws_ref, wn_ref, b_ref, hs2_ref, hn2_ref):
        agg = ag_ref[0] + ag_ref[1]
        deg = dg_ref[0, :, 0:1] + dg_ref[1, :, 0:1]
        invd = 1.0 / jnp.maximum(deg, 1.0)
        h = jnp.maximum(hs_ref[...] + agg * invd, 0.0)
        hs2_ref[...] = jnp.dot(h, ws_ref[...], preferred_element_type=jnp.float32) + b_ref[...]
        if emit_h:
            hn2_ref[...] = h
        else:
            hn2_ref[...] = jnp.dot(h, wn_ref[...], preferred_element_type=jnp.float32)

    w = Ws.shape[1]
    return pl.pallas_call(
        body,
        grid=(N_PAD // BM,),
        in_specs=[
            pl.BlockSpec((BM, FEAT), lambda i: (i, 0)),
            pl.BlockSpec((NUM_SC, BM, FEAT), lambda i: (0, i, 0)),
            pl.BlockSpec((NUM_SC, BM, 16), lambda i: (0, i, 0)),
            pl.BlockSpec((FEAT, w), lambda i: (0, 0)),
            pl.BlockSpec((FEAT, FEAT), lambda i: (0, 0)),
            pl.BlockSpec((1, w), lambda i: (0, 0)),
        ],
        out_specs=[
            pl.BlockSpec((BM, w), lambda i: (i, 0)),
            pl.BlockSpec((BM, FEAT), lambda i: (i, 0)),
        ],
        out_shape=[
            jax.ShapeDtypeStruct((N_PAD, w), jnp.float32),
            jax.ShapeDtypeStruct((N_PAD, FEAT), jnp.float32),
        ],
    )(hs_prev, aggp, degp, Ws, Wn, b.reshape(1, w))


def _final_combine(hs3, aggp, degp, Wn):
    def body(hs_ref, ag_ref, dg_ref, wn_ref, o_ref):
        agg = ag_ref[0] + ag_ref[1]
        deg = dg_ref[0, :, 0:1] + dg_ref[1, :, 0:1]
        invd = 1.0 / jnp.maximum(deg, 1.0)
        hn = agg * invd
        o_ref[...] = hs_ref[...] + jnp.dot(hn, wn_ref[...], preferred_element_type=jnp.float32)

    return pl.pallas_call(
        body,
        grid=(N_PAD // BM,),
        in_specs=[
            pl.BlockSpec((BM, CP), lambda i: (i, 0)),
            pl.BlockSpec((NUM_SC, BM, FEAT), lambda i: (0, i, 0)),
            pl.BlockSpec((NUM_SC, BM, 16), lambda i: (0, i, 0)),
            pl.BlockSpec((FEAT, CP), lambda i: (0, 0)),
        ],
        out_specs=pl.BlockSpec((BM, CP), lambda i: (i, 0)),
        out_shape=jax.ShapeDtypeStruct((N_PAD, CP), jnp.float32),
    )(hs3, aggp, degp, Wn)


def kernel(x, edge_index, Ws1, Wn1, b1, Ws2, Wn2, b2, Ws3, Wn3, b3):
    src = edge_index[0].astype(jnp.int32)
    dst = edge_index[1].astype(jnp.int32)
    pad = E_PAD - N_EDGES
    srcr = jnp.concatenate([src, jnp.zeros((pad,), jnp.int32)]).reshape(
        NUM_SC * NSUB, NGRP, GRP, KCH)
    dstr = jnp.concatenate([dst, jnp.full((pad,), N_NODES, jnp.int32)]).reshape(
        NUM_SC * NSUB, NGRP, GRP, KCH)
    xp = jnp.pad(x, ((0, N_PAD - N_NODES), (0, 0)))
    Ws3p = jnp.pad(Ws3, ((0, 0), (0, CP - N_CLS)))
    Wn3p = jnp.pad(Wn3, ((0, 0), (0, CP - N_CLS)))
    b3p = jnp.pad(b3, (0, CP - N_CLS))

    degp = _sc_degree(dstr)
    hs1, hn1 = _dense(xp, Ws1, Wn1, b1)
    agg1 = _sc_aggregate(hn1, srcr, dstr)
    hs2, hn2 = _combine_dense(hs1, agg1, degp, Ws2, Wn2, b2, emit_h=False)
    agg2 = _sc_aggregate(hn2, srcr, dstr)
    hs3, h2 = _combine_dense(hs2, agg2, degp, Ws3p, Wn3p, b3p, emit_h=True)
    agg3 = _sc_aggregate(h2, srcr, dstr)
    out = _final_combine(hs3, agg3, degp, Wn3p)
    return out[:N_NODES, :N_CLS]


# speedup vs baseline: 3.2642x; 1.0945x over previous
import jax
import jax.numpy as jnp
from jax import lax
from jax.experimental import pallas as pl
from jax.experimental.pallas import tpu as pltpu
from jax.experimental.pallas import tpu_sc as plsc

N_NODES = 10000
FEAT = 128
N_CLS = 47
CP = 48
N_PAD = 10240
N_EDGES = 320000
NUM_SC = 2
NSUB = 16
KCH = 128
GRP = 8
NGRP = 10
E_PAD = NUM_SC * NSUB * NGRP * GRP * KCH
ROWS_PER_TILE = N_PAD // NSUB
BM = 1024


def _sc_aggregate(table, srcr, dstr):
    def body(table_h, src_h, dst_h, out_h, src_v, dst_v, gbuf0, gbuf1, acc_sh,
             gsem0, gsem1, ssem0, ssem1):
        cid = lax.axis_index("c")
        sid = lax.axis_index("s")
        wid = cid * NSUB + sid

        zv = jnp.zeros((16,), jnp.float32)

        @pl.loop(0, KCH)
        def _(r):
            for c2 in range(FEAT // 16):
                gbuf0[r, pl.ds(c2 * 16, 16)] = zv

        base = sid * ROWS_PER_TILE
        for j in range(ROWS_PER_TILE // KCH):
            pltpu.sync_copy(gbuf0, acc_sh.at[pl.ds(base + j * KCH, KCH)])
        plsc.subcore_barrier()

        bufs = (gbuf0, gbuf1)
        gsems = (gsem0, gsem1)
        ssems = (ssem0, ssem1)

        @pl.loop(0, NGRP)
        def _(g):
            pltpu.sync_copy(src_h.at[wid, g], src_v)
            pltpu.sync_copy(dst_h.at[wid, g], dst_v)
            dg = [
                pltpu.async_copy(table_h.at[src_v.at[0]], gbuf0, gsem0),
                pltpu.async_copy(table_h.at[src_v.at[1]], gbuf1, gsem1),
            ]
            for j in range(GRP):
                p = j % 2
                dg[p].wait()
                ds = pltpu.async_copy(bufs[p], acc_sh.at[dst_v.at[j]], ssems[p],
                                      add=True)
                ds.wait()
                if j + 2 < GRP:
                    dg[p] = pltpu.async_copy(table_h.at[src_v.at[j + 2]], bufs[p],
                                             gsems[p])

        plsc.subcore_barrier()

        for j in range(ROWS_PER_TILE // KCH):
            r0 = base + j * KCH
            pltpu.sync_copy(acc_sh.at[pl.ds(r0, KCH)], out_h.at[cid, pl.ds(r0, KCH)])

    mesh = plsc.VectorSubcoreMesh(core_axis_name="c", subcore_axis_name="s")
    k = pl.kernel(
        body,
        out_type=jax.ShapeDtypeStruct((NUM_SC, N_PAD, FEAT), jnp.float32),
        mesh=mesh,
        scratch_types=[
            pltpu.VMEM((GRP, KCH), jnp.int32),
            pltpu.VMEM((GRP, KCH), jnp.int32),
            pltpu.VMEM((KCH, FEAT), jnp.float32),
            pltpu.VMEM((KCH, FEAT), jnp.float32),
            pltpu.VMEM_SHARED((N_PAD, FEAT), jnp.float32),
            pltpu.SemaphoreType.DMA,
            pltpu.SemaphoreType.DMA,
            pltpu.SemaphoreType.DMA,
            pltpu.SemaphoreType.DMA,
        ])
    return k(table, srcr, dstr)


def _sc_degree(dstr):
    def body(dst_h, deg_h, dst_v, ones_v, deg_sh):
        cid = lax.axis_index("c")
        sid = lax.axis_index("s")
        wid = cid * NSUB + sid
        zv = jnp.zeros((16,), jnp.float32)

        @pl.loop(0, KCH)
        def _(r):
            ones_v[r, :] = zv

        base = sid * ROWS_PER_TILE
        for j in range(ROWS_PER_TILE // KCH):
            pltpu.sync_copy(ones_v, deg_sh.at[pl.ds(base + j * KCH, KCH)])
        ov = jnp.ones((16,), jnp.float32)

        @pl.loop(0, KCH)
        def _(r):
            ones_v[r, :] = ov

        plsc.subcore_barrier()

        @pl.loop(0, NGRP)
        def _(g):
            pltpu.sync_copy(dst_h.at[wid, g], dst_v)
            for j in range(GRP):
                pltpu.sync_copy(ones_v, deg_sh.at[dst_v.at[j]], add=True)

        plsc.subcore_barrier()
        for j in range(ROWS_PER_TILE // KCH):
            r0 = base + j * KCH
            pltpu.sync_copy(deg_sh.at[pl.ds(r0, KCH)], deg_h.at[cid, pl.ds(r0, KCH)])

    mesh = plsc.VectorSubcoreMesh(core_axis_name="c", subcore_axis_name="s")
    k = pl.kernel(
        body,
        out_type=jax.ShapeDtypeStruct((NUM_SC, N_PAD, 16), jnp.float32),
        mesh=mesh,
        scratch_types=[
            pltpu.VMEM((GRP, KCH), jnp.int32),
            pltpu.VMEM((KCH, 16), jnp.float32),
            pltpu.VMEM_SHARED((N_PAD, 16), jnp.float32),
        ],
        compiler_params=pltpu.CompilerParams(use_tc_tiling_on_sc=False))
    return k(dstr)


def _dense(x, Ws, Wn, b):
    def body(x_ref, ws_ref, wn_ref, b_ref, hs_ref, hn_ref):
        xb = x_ref[...]
        hs_ref[...] = jnp.dot(xb, ws_ref[...], preferred_element_type=jnp.float32) + b_ref[...]
        hn_ref[...] = jnp.dot(xb, wn_ref[...], preferred_element_type=jnp.float32)

    return pl.pallas_call(
        body,
        grid=(N_PAD // BM,),
        in_specs=[
            pl.BlockSpec((BM, FEAT), lambda i: (i, 0)),
            pl.BlockSpec((FEAT, FEAT), lambda i: (0, 0)),
            pl.BlockSpec((FEAT, FEAT), lambda i: (0, 0)),
            pl.BlockSpec((1, FEAT), lambda i: (0, 0)),
        ],
        out_specs=[pl.BlockSpec((BM, FEAT), lambda i: (i, 0))] * 2,
        out_shape=[jax.ShapeDtypeStruct((N_PAD, FEAT), jnp.float32)] * 2,
    )(x, Ws, Wn, b.reshape(1, FEAT))


def _combine_dense(hs_prev, aggp, degp, Ws, Wn, b, emit_h):
    def body(hs_ref, ag_ref, dg_ref, ws_ref, wn_ref, b_ref, hs2_ref, hn2_ref):
        agg = ag_ref[0] + ag_ref[1]
        deg = dg_ref[0, :, 0:1] + dg_ref[1, :, 0:1]
        invd = 1.0 / jnp.maximum(deg, 1.0)
        h = jnp.maximum(hs_ref[...] + agg * invd, 0.0)
        hs2_ref[...] = jnp.dot(h, ws_ref[...], preferred_element_type=jnp.float32) + b_ref[...]
        if emit_h:
            hn2_ref[...] = h
        else:
            hn2_ref[...] = jnp.dot(h, wn_ref[...], preferred_element_type=jnp.float32)

    w = Ws.shape[1]
    return pl.pallas_call(
        body,
        grid=(N_PAD // BM,),
        in_specs=[
            pl.BlockSpec((BM, FEAT), lambda i: (i, 0)),
            pl.BlockSpec((NUM_SC, BM, FEAT), lambda i: (0, i, 0)),
            pl.BlockSpec((NUM_SC, BM, 16), lambda i: (0, i, 0)),
            pl.BlockSpec((FEAT, w), lambda i: (0, 0)),
            pl.BlockSpec((FEAT, FEAT), lambda i: (0, 0)),
            pl.BlockSpec((1, w), lambda i: (0, 0)),
        ],
        out_specs=[
            pl.BlockSpec((BM, w), lambda i: (i, 0)),
            pl.BlockSpec((BM, FEAT), lambda i: (i, 0)),
        ],
        out_shape=[
            jax.ShapeDtypeStruct((N_PAD, w), jnp.float32),
            jax.ShapeDtypeStruct((N_PAD, FEAT), jnp.float32),
        ],
    )(hs_prev, aggp, degp, Ws, Wn, b.reshape(1, w))


def _final_combine(hs3, aggp, degp, Wn):
    def body(hs_ref, ag_ref, dg_ref, wn_ref, o_ref):
        agg = ag_ref[0] + ag_ref[1]
        deg = dg_ref[0, :, 0:1] + dg_ref[1, :, 0:1]
        invd = 1.0 / jnp.maximum(deg, 1.0)
        hn = agg * invd
        o_ref[...] = hs_ref[...] + jnp.dot(hn, wn_ref[...], preferred_element_type=jnp.float32)

    return pl.pallas_call(
        body,
        grid=(N_PAD // BM,),
        in_specs=[
            pl.BlockSpec((BM, CP), lambda i: (i, 0)),
            pl.BlockSpec((NUM_SC, BM, FEAT), lambda i: (0, i, 0)),
            pl.BlockSpec((NUM_SC, BM, 16), lambda i: (0, i, 0)),
            pl.BlockSpec((FEAT, CP), lambda i: (0, 0)),
        ],
        out_specs=pl.BlockSpec((BM, CP), lambda i: (i, 0)),
        out_shape=jax.ShapeDtypeStruct((N_PAD, CP), jnp.float32),
    )(hs3, aggp, degp, Wn)


def kernel(x, edge_index, Ws1, Wn1, b1, Ws2, Wn2, b2, Ws3, Wn3, b3):
    src = edge_index[0].astype(jnp.int32)
    dst = edge_index[1].astype(jnp.int32)
    pad = E_PAD - N_EDGES
    srcr = jnp.concatenate([src, jnp.zeros((pad,), jnp.int32)]).reshape(
        NUM_SC * NSUB, NGRP, GRP, KCH)
    dstr = jnp.concatenate([dst, jnp.full((pad,), N_NODES, jnp.int32)]).reshape(
        NUM_SC * NSUB, NGRP, GRP, KCH)
    xp = jnp.pad(x, ((0, N_PAD - N_NODES), (0, 0)))
    Ws3p = jnp.pad(Ws3, ((0, 0), (0, CP - N_CLS)))
    Wn3p = jnp.pad(Wn3, ((0, 0), (0, CP - N_CLS)))
    b3p = jnp.pad(b3, (0, CP - N_CLS))

    degp = _sc_degree(dstr)
    hs1, hn1 = _dense(xp, Ws1, Wn1, b1)
    agg1 = _sc_aggregate(hn1, srcr, dstr)
    hs2, hn2 = _combine_dense(hs1, agg1, degp, Ws2, Wn2, b2, emit_h=False)
    agg2 = _sc_aggregate(hn2, srcr, dstr)
    hs3, h2 = _combine_dense(hs2, agg2, degp, Ws3p, Wn3p, b3p, emit_h=True)
    agg3 = _sc_aggregate(h2, srcr, dstr)
    out = _final_combine(hs3, agg3, degp, Wn3p)
    return out[:N_NODES, :N_CLS]


# speedup vs baseline: 10.9032x; 3.3403x over previous
import jax
import jax.numpy as jnp
from jax import lax
from jax.experimental import pallas as pl
from jax.experimental.pallas import tpu as pltpu
from jax.experimental.pallas import tpu_sc as plsc

N_NODES = 10000
FEAT = 128
N_CLS = 47
CP = 48
N_PAD = 10240
N_EDGES = 320000
NUM_SC = 2
NSUB = 16
KCH = 128
GRP = 8
NGRP = 10
E_PAD = NUM_SC * NSUB * NGRP * GRP * KCH
ROWS_PER_TILE = N_PAD // NSUB
BM = 1024


def _sc_aggregate(table, srcr, dstr):
    def body(table_h, src_h, dst_h, out_h, src_v, dst_v, gbuf0, gbuf1, acc_sh,
             gsem0, gsem1, ssem0, ssem1):
        cid = lax.axis_index("c")
        sid = lax.axis_index("s")
        wid = cid * NSUB + sid

        zv = jnp.zeros((16,), jnp.float32)

        @pl.loop(0, KCH)
        def _(r):
            for c2 in range(FEAT // 16):
                gbuf0[r, pl.ds(c2 * 16, 16)] = zv

        base = sid * ROWS_PER_TILE
        for j in range(ROWS_PER_TILE // KCH):
            pltpu.sync_copy(gbuf0, acc_sh.at[pl.ds(base + j * KCH, KCH)])
        plsc.subcore_barrier()

        bufs = (gbuf0, gbuf1)
        gsems = (gsem0, gsem1)
        ssems = (ssem0, ssem1)

        @pl.loop(0, NGRP)
        def _(g):
            pltpu.sync_copy(src_h.at[wid, g], src_v)
            pltpu.sync_copy(dst_h.at[wid, g], dst_v)
            dg = [
                pltpu.async_copy(table_h.at[src_v.at[0]], gbuf0, gsem0),
                pltpu.async_copy(table_h.at[src_v.at[1]], gbuf1, gsem1),
            ]
            for j in range(GRP):
                p = j % 2
                dg[p].wait()
                ds = pltpu.async_copy(bufs[p], acc_sh.at[dst_v.at[j]], ssems[p],
                                      add=True)
                ds.wait()
                if j + 2 < GRP:
                    dg[p] = pltpu.async_copy(table_h.at[src_v.at[j + 2]], bufs[p],
                                             gsems[p])

        plsc.subcore_barrier()

        for j in range(ROWS_PER_TILE // KCH):
            r0 = base + j * KCH
            pltpu.sync_copy(acc_sh.at[pl.ds(r0, KCH)], out_h.at[cid, pl.ds(r0, KCH)])

    mesh = plsc.VectorSubcoreMesh(core_axis_name="c", subcore_axis_name="s")
    k = pl.kernel(
        body,
        out_type=jax.ShapeDtypeStruct((NUM_SC, N_PAD, FEAT), jnp.float32),
        mesh=mesh,
        scratch_types=[
            pltpu.VMEM((GRP, KCH), jnp.int32),
            pltpu.VMEM((GRP, KCH), jnp.int32),
            pltpu.VMEM((KCH, FEAT), jnp.float32),
            pltpu.VMEM((KCH, FEAT), jnp.float32),
            pltpu.VMEM_SHARED((N_PAD, FEAT), jnp.float32),
            pltpu.SemaphoreType.DMA,
            pltpu.SemaphoreType.DMA,
            pltpu.SemaphoreType.DMA,
            pltpu.SemaphoreType.DMA,
        ])
    return k(table, srcr, dstr)


def _sc_degree(dstr):
    def body(dst_h, deg_h, dst_v, ones_v, deg_sh):
        cid = lax.axis_index("c")
        sid = lax.axis_index("s")
        wid = cid * NSUB + sid
        zv = jnp.zeros((16,), jnp.float32)

        @pl.loop(0, KCH)
        def _(r):
            ones_v[r, :] = zv

        base = sid * ROWS_PER_TILE
        for j in range(ROWS_PER_TILE // KCH):
            pltpu.sync_copy(ones_v, deg_sh.at[pl.ds(base + j * KCH, KCH)])
        ov = jnp.ones((16,), jnp.float32)

        @pl.loop(0, KCH)
        def _(r):
            ones_v[r, :] = ov

        plsc.subcore_barrier()

        @pl.loop(0, NGRP)
        def _(g):
            pltpu.sync_copy(dst_h.at[wid, g], dst_v)
            for j in range(GRP):
                pltpu.sync_copy(ones_v, deg_sh.at[dst_v.at[j]], add=True)

        plsc.subcore_barrier()
        for j in range(ROWS_PER_TILE // KCH):
            r0 = base + j * KCH
            pltpu.sync_copy(deg_sh.at[pl.ds(r0, KCH)], deg_h.at[cid, pl.ds(r0, KCH)])

    mesh = plsc.VectorSubcoreMesh(core_axis_name="c", subcore_axis_name="s")
    k = pl.kernel(
        body,
        out_type=jax.ShapeDtypeStruct((NUM_SC, N_PAD, 16), jnp.float32),
        mesh=mesh,
        scratch_types=[
            pltpu.VMEM((GRP, KCH), jnp.int32),
            pltpu.VMEM((KCH, 16), jnp.float32),
            pltpu.VMEM_SHARED((N_PAD, 16), jnp.float32),
        ],
        compiler_params=pltpu.CompilerParams(use_tc_tiling_on_sc=False))
    return k(dstr)


def _dense(x, Ws, Wn, b):
    def body(x_ref, ws_ref, wn_ref, b_ref, hs_ref, hn_ref):
        xb = x_ref[...]
        hs_ref[...] = jnp.dot(xb, ws_ref[...], preferred_element_type=jnp.float32) + b_ref[...]
        hn_ref[...] = jnp.dot(xb, wn_ref[...], preferred_element_type=jnp.float32)

    return pl.pallas_call(
        body,
        grid=(N_PAD // BM,),
        in_specs=[
            pl.BlockSpec((BM, FEAT), lambda i: (i, 0)),
            pl.BlockSpec((FEAT, FEAT), lambda i: (0, 0)),
            pl.BlockSpec((FEAT, FEAT), lambda i: (0, 0)),
            pl.BlockSpec((1, FEAT), lambda i: (0, 0)),
        ],
        out_specs=[pl.BlockSpec((BM, FEAT), lambda i: (i, 0))] * 2,
        out_shape=[jax.ShapeDtypeStruct((N_PAD, FEAT), jnp.float32)] * 2,
    )(x, Ws, Wn, b.reshape(1, FEAT))


def _combine_dense(hs_prev, aggp, degp, Ws, Wn, b, emit_h):
    def body(hs_ref, ag_ref, dg_ref, ws_ref, wn_ref, b_ref, hs2_ref, hn2_ref):
        agg = ag_ref[0] + ag_ref[1]
        deg = dg_ref[0, :, 0:1] + dg_ref[1, :, 0:1]
        invd = 1.0 / jnp.maximum(deg, 1.0)
        h = jnp.maximum(hs_ref[...] + agg * invd, 0.0)
        hs2_ref[...] = jnp.dot(h, ws_ref[...], preferred_element_type=jnp.float32) + b_ref[...]
        if emit_h:
            hn2_ref[...] = h
        else:
            hn2_ref[...] = jnp.dot(h, wn_ref[...], preferred_element_type=jnp.float32)

    w = Ws.shape[1]
    return pl.pallas_call(
        body,
        grid=(N_PAD // BM,),
        in_specs=[
            pl.BlockSpec((BM, FEAT), lambda i: (i, 0)),
            pl.BlockSpec((NUM_SC, BM, FEAT), lambda i: (0, i, 0)),
            pl.BlockSpec((NUM_SC, BM, 16), lambda i: (0, i, 0)),
            pl.BlockSpec((FEAT, w), lambda i: (0, 0)),
            pl.BlockSpec((FEAT, FEAT), lambda i: (0, 0)),
            pl.BlockSpec((1, w), lambda i: (0, 0)),
        ],
        out_specs=[
            pl.BlockSpec((BM, w), lambda i: (i, 0)),
            pl.BlockSpec((BM, FEAT), lambda i: (i, 0)),
        ],
        out_shape=[
            jax.ShapeDtypeStruct((N_PAD, w), jnp.float32),
            jax.ShapeDtypeStruct((N_PAD, FEAT), jnp.float32),
        ],
    )(hs_prev, aggp, degp, Ws, Wn, b.reshape(1, w))


def _final_combine(hs3, aggp, degp, Wn):
    def body(hs_ref, ag_ref, dg_ref, wn_ref, o_ref):
        agg = ag_ref[0] + ag_ref[1]
        deg = dg_ref[0, :, 0:1] + dg_ref[1, :, 0:1]
        invd = 1.0 / jnp.maximum(deg, 1.0)
        hn = agg * invd
        o_ref[...] = hs_ref[...] + jnp.dot(hn, wn_ref[...], preferred_element_type=jnp.float32)

    return pl.pallas_call(
        body,
        grid=(N_PAD // BM,),
        in_specs=[
            pl.BlockSpec((BM, CP), lambda i: (i, 0)),
            pl.BlockSpec((NUM_SC, BM, FEAT), lambda i: (0, i, 0)),
            pl.BlockSpec((NUM_SC, BM, 16), lambda i: (0, i, 0)),
            pl.BlockSpec((FEAT, CP), lambda i: (0, 0)),
        ],
        out_specs=pl.BlockSpec((BM, CP), lambda i: (i, 0)),
        out_shape=jax.ShapeDtypeStruct((N_PAD, CP), jnp.float32),
    )(hs3, aggp, degp, Wn)


def kernel(x, edge_index, Ws1, Wn1, b1, Ws2, Wn2, b2, Ws3, Wn3, b3):
    src = edge_index[0].astype(jnp.int32)
    dst = edge_index[1].astype(jnp.int32)
    pad = E_PAD - N_EDGES
    pad_ids = jnp.arange(pad, dtype=jnp.int32)
    srcr = jnp.concatenate([src, pad_ids % N_NODES]).reshape(
        NUM_SC * NSUB, NGRP, GRP, KCH)
    dstr = jnp.concatenate([dst, N_NODES + pad_ids % (N_PAD - N_NODES)]).reshape(
        NUM_SC * NSUB, NGRP, GRP, KCH)
    xp = jnp.pad(x, ((0, N_PAD - N_NODES), (0, 0)))
    Ws3p = jnp.pad(Ws3, ((0, 0), (0, CP - N_CLS)))
    Wn3p = jnp.pad(Wn3, ((0, 0), (0, CP - N_CLS)))
    b3p = jnp.pad(b3, (0, CP - N_CLS))

    degp = _sc_degree(dstr)
    hs1, hn1 = _dense(xp, Ws1, Wn1, b1)
    agg1 = _sc_aggregate(hn1, srcr, dstr)
    hs2, hn2 = _combine_dense(hs1, agg1, degp, Ws2, Wn2, b2, emit_h=False)
    agg2 = _sc_aggregate(hn2, srcr, dstr)
    hs3, h2 = _combine_dense(hs2, agg2, degp, Ws3p, Wn3p, b3p, emit_h=True)
    agg3 = _sc_aggregate(h2, srcr, dstr)
    out = _final_combine(hs3, agg3, degp, Wn3p)
    return out[:N_NODES, :N_CLS]


# speedup vs baseline: 11.8300x; 1.0850x over previous
import jax
import jax.numpy as jnp
from jax import lax
from jax.experimental import pallas as pl
from jax.experimental.pallas import tpu as pltpu
from jax.experimental.pallas import tpu_sc as plsc

N_NODES = 10000
FEAT = 128
N_CLS = 47
CP = 48
N_PAD = 10240
N_EDGES = 320000
NUM_SC = 2
NSUB = 16
KCH = 128
GRP = 8
NGRP = 10
E_PAD = NUM_SC * NSUB * NGRP * GRP * KCH
ROWS_PER_TILE = N_PAD // NSUB
BM = 1024


def _sc_aggregate(table, srcr, dstr, width=FEAT):
    def body(table_h, src_h, dst_h, out_h, src_v, dst_v, gbuf0, gbuf1, acc_sh,
             gsem0, gsem1, ssem0, ssem1):
        cid = lax.axis_index("c")
        sid = lax.axis_index("s")
        wid = cid * NSUB + sid

        zv = jnp.zeros((16,), jnp.float32)

        @pl.loop(0, KCH)
        def _(r):
            for c2 in range(width // 16):
                gbuf0[r, pl.ds(c2 * 16, 16)] = zv

        base = sid * ROWS_PER_TILE
        for j in range(ROWS_PER_TILE // KCH):
            pltpu.sync_copy(gbuf0, acc_sh.at[pl.ds(base + j * KCH, KCH)])
        plsc.subcore_barrier()

        bufs = (gbuf0, gbuf1)
        gsems = (gsem0, gsem1)
        ssems = (ssem0, ssem1)

        @pl.loop(0, NGRP)
        def _(g):
            pltpu.sync_copy(src_h.at[wid, g], src_v)
            pltpu.sync_copy(dst_h.at[wid, g], dst_v)
            dg = [
                pltpu.async_copy(table_h.at[src_v.at[0]], gbuf0, gsem0),
                pltpu.async_copy(table_h.at[src_v.at[1]], gbuf1, gsem1),
            ]
            for j in range(GRP):
                p = j % 2
                dg[p].wait()
                ds = pltpu.async_copy(bufs[p], acc_sh.at[dst_v.at[j]], ssems[p],
                                      add=True)
                ds.wait()
                if j + 2 < GRP:
                    dg[p] = pltpu.async_copy(table_h.at[src_v.at[j + 2]], bufs[p],
                                             gsems[p])

        plsc.subcore_barrier()

        for j in range(ROWS_PER_TILE // KCH):
            r0 = base + j * KCH
            pltpu.sync_copy(acc_sh.at[pl.ds(r0, KCH)], out_h.at[cid, pl.ds(r0, KCH)])

    mesh = plsc.VectorSubcoreMesh(core_axis_name="c", subcore_axis_name="s")
    params = None
    if width % 128 != 0:
        params = pltpu.CompilerParams(use_tc_tiling_on_sc=False)
    k = pl.kernel(
        body,
        out_type=jax.ShapeDtypeStruct((NUM_SC, N_PAD, width), jnp.float32),
        mesh=mesh,
        scratch_types=[
            pltpu.VMEM((GRP, KCH), jnp.int32),
            pltpu.VMEM((GRP, KCH), jnp.int32),
            pltpu.VMEM((KCH, width), jnp.float32),
            pltpu.VMEM((KCH, width), jnp.float32),
            pltpu.VMEM_SHARED((N_PAD, width), jnp.float32),
            pltpu.SemaphoreType.DMA,
            pltpu.SemaphoreType.DMA,
            pltpu.SemaphoreType.DMA,
            pltpu.SemaphoreType.DMA,
        ],
        compiler_params=params)
    return k(table, srcr, dstr)


def _sc_degree(dstr):
    def body(dst_h, deg_h, dst_v, ones_v, deg_sh):
        cid = lax.axis_index("c")
        sid = lax.axis_index("s")
        wid = cid * NSUB + sid
        zv = jnp.zeros((16,), jnp.float32)

        @pl.loop(0, KCH)
        def _(r):
            ones_v[r, :] = zv

        base = sid * ROWS_PER_TILE
        for j in range(ROWS_PER_TILE // KCH):
            pltpu.sync_copy(ones_v, deg_sh.at[pl.ds(base + j * KCH, KCH)])
        ov = jnp.ones((16,), jnp.float32)

        @pl.loop(0, KCH)
        def _(r):
            ones_v[r, :] = ov

        plsc.subcore_barrier()

        @pl.loop(0, NGRP)
        def _(g):
            pltpu.sync_copy(dst_h.at[wid, g], dst_v)
            for j in range(GRP):
                pltpu.sync_copy(ones_v, deg_sh.at[dst_v.at[j]], add=True)

        plsc.subcore_barrier()
        for j in range(ROWS_PER_TILE // KCH):
            r0 = base + j * KCH
            pltpu.sync_copy(deg_sh.at[pl.ds(r0, KCH)], deg_h.at[cid, pl.ds(r0, KCH)])

    mesh = plsc.VectorSubcoreMesh(core_axis_name="c", subcore_axis_name="s")
    k = pl.kernel(
        body,
        out_type=jax.ShapeDtypeStruct((NUM_SC, N_PAD, 16), jnp.float32),
        mesh=mesh,
        scratch_types=[
            pltpu.VMEM((GRP, KCH), jnp.int32),
            pltpu.VMEM((KCH, 16), jnp.float32),
            pltpu.VMEM_SHARED((N_PAD, 16), jnp.float32),
        ],
        compiler_params=pltpu.CompilerParams(use_tc_tiling_on_sc=False))
    return k(dstr)


def _dense(x, Ws, Wn, b):
    def body(x_ref, ws_ref, wn_ref, b_ref, hs_ref, hn_ref):
        xb = x_ref[...]
        hs_ref[...] = jnp.dot(xb, ws_ref[...], preferred_element_type=jnp.float32) + b_ref[...]
        hn_ref[...] = jnp.dot(xb, wn_ref[...], preferred_element_type=jnp.float32)

    return pl.pallas_call(
        body,
        grid=(N_PAD // BM,),
        in_specs=[
            pl.BlockSpec((BM, FEAT), lambda i: (i, 0)),
            pl.BlockSpec((FEAT, FEAT), lambda i: (0, 0)),
            pl.BlockSpec((FEAT, FEAT), lambda i: (0, 0)),
            pl.BlockSpec((1, FEAT), lambda i: (0, 0)),
        ],
        out_specs=[pl.BlockSpec((BM, FEAT), lambda i: (i, 0))] * 2,
        out_shape=[jax.ShapeDtypeStruct((N_PAD, FEAT), jnp.float32)] * 2,
    )(x, Ws, Wn, b.reshape(1, FEAT))


def _combine_dense(hs_prev, aggp, degp, Ws, Wn, b):
    def body(hs_ref, ag_ref, dg_ref, ws_ref, wn_ref, b_ref, hs2_ref, hn2_ref):
        agg = ag_ref[0] + ag_ref[1]
        deg = dg_ref[0, :, 0:1] + dg_ref[1, :, 0:1]
        invd = 1.0 / jnp.maximum(deg, 1.0)
        h = jnp.maximum(hs_ref[...] + agg * invd, 0.0)
        hs2_ref[...] = jnp.dot(h, ws_ref[...], preferred_element_type=jnp.float32) + b_ref[...]
        hn2_ref[...] = jnp.dot(h, wn_ref[...], preferred_element_type=jnp.float32)

    ws_w = Ws.shape[1]
    wn_w = Wn.shape[1]
    return pl.pallas_call(
        body,
        grid=(N_PAD // BM,),
        in_specs=[
            pl.BlockSpec((BM, FEAT), lambda i: (i, 0)),
            pl.BlockSpec((NUM_SC, BM, FEAT), lambda i: (0, i, 0)),
            pl.BlockSpec((NUM_SC, BM, 16), lambda i: (0, i, 0)),
            pl.BlockSpec((FEAT, ws_w), lambda i: (0, 0)),
            pl.BlockSpec((FEAT, wn_w), lambda i: (0, 0)),
            pl.BlockSpec((1, ws_w), lambda i: (0, 0)),
        ],
        out_specs=[
            pl.BlockSpec((BM, ws_w), lambda i: (i, 0)),
            pl.BlockSpec((BM, wn_w), lambda i: (i, 0)),
        ],
        out_shape=[
            jax.ShapeDtypeStruct((N_PAD, ws_w), jnp.float32),
            jax.ShapeDtypeStruct((N_PAD, wn_w), jnp.float32),
        ],
    )(hs_prev, aggp, degp, Ws, Wn, b.reshape(1, ws_w))


def _final_combine(hs3, aggp, degp):
    def body(hs_ref, ag_ref, dg_ref, o_ref):
        agg = ag_ref[0] + ag_ref[1]
        deg = dg_ref[0, :, 0:1] + dg_ref[1, :, 0:1]
        invd = 1.0 / jnp.maximum(deg, 1.0)
        o_ref[...] = hs_ref[...] + agg * invd

    return pl.pallas_call(
        body,
        grid=(N_PAD // BM,),
        in_specs=[
            pl.BlockSpec((BM, CP), lambda i: (i, 0)),
            pl.BlockSpec((NUM_SC, BM, CP), lambda i: (0, i, 0)),
            pl.BlockSpec((NUM_SC, BM, 16), lambda i: (0, i, 0)),
        ],
        out_specs=pl.BlockSpec((BM, CP), lambda i: (i, 0)),
        out_shape=jax.ShapeDtypeStruct((N_PAD, CP), jnp.float32),
    )(hs3, aggp, degp)


def kernel(x, edge_index, Ws1, Wn1, b1, Ws2, Wn2, b2, Ws3, Wn3, b3):
    src = edge_index[0].astype(jnp.int32)
    dst = edge_index[1].astype(jnp.int32)
    pad = E_PAD - N_EDGES
    pad_ids = jnp.arange(pad, dtype=jnp.int32)
    srcr = jnp.concatenate([src, pad_ids % N_NODES]).reshape(
        NUM_SC * NSUB, NGRP, GRP, KCH)
    dstr = jnp.concatenate([dst, N_NODES + pad_ids % (N_PAD - N_NODES)]).reshape(
        NUM_SC * NSUB, NGRP, GRP, KCH)
    xp = jnp.pad(x, ((0, N_PAD - N_NODES), (0, 0)))
    Ws3p = jnp.pad(Ws3, ((0, 0), (0, CP - N_CLS)))
    Wn3p = jnp.pad(Wn3, ((0, 0), (0, CP - N_CLS)))
    b3p = jnp.pad(b3, (0, CP - N_CLS))

    degp = _sc_degree(dstr)
    hs1, hn1 = _dense(xp, Ws1, Wn1, b1)
    agg1 = _sc_aggregate(hn1, srcr, dstr)
    hs2, hn2 = _combine_dense(hs1, agg1, degp, Ws2, Wn2, b2)
    agg2 = _sc_aggregate(hn2, srcr, dstr)
    hs3, hn3 = _combine_dense(hs2, agg2, degp, Ws3p, Wn3p, b3p)
    agg3 = _sc_aggregate(hn3, srcr, dstr, width=CP)
    out = _final_combine(hs3, agg3, degp)
    return out[:N_NODES, :N_CLS]


# speedup vs baseline: 13.0392x; 1.1022x over previous
import jax
import jax.numpy as jnp
from jax import lax
from jax.experimental import pallas as pl
from jax.experimental.pallas import tpu as pltpu
from jax.experimental.pallas import tpu_sc as plsc

N_NODES = 10000
FEAT = 128
N_CLS = 47
CP = 48
N_PAD = 10240
N_EDGES = 320000
NUM_SC = 2
NSUB = 16
KCH = 128
GRP = 8
NGRP = 10
E_PAD = NUM_SC * NSUB * NGRP * GRP * KCH
ROWS_PER_TILE = N_PAD // NSUB
BM = 1024


def _sc_aggregate(table, srcr, dstr, width=FEAT):
    def body(table_h, src_h, dst_h, out_h, src_v, dst_v, gbuf0, gbuf1, acc_sh,
             gsem0, gsem1, ssem0, ssem1, isem0, isem1):
        cid = lax.axis_index("c")
        sid = lax.axis_index("s")
        wid = cid * NSUB + sid

        zv = jnp.zeros((16,), jnp.float32)

        @pl.loop(0, KCH)
        def _(r):
            for c2 in range(width // 16):
                gbuf0[r, pl.ds(c2 * 16, 16)] = zv

        base = sid * ROWS_PER_TILE
        for j in range(ROWS_PER_TILE // KCH):
            pltpu.sync_copy(gbuf0, acc_sh.at[pl.ds(base + j * KCH, KCH)])
        plsc.subcore_barrier()

        bufs = (gbuf0, gbuf1)
        gsems = (gsem0, gsem1)
        ssems = (ssem0, ssem1)
        isems = (isem0, isem1)

        pltpu.async_copy(src_h.at[wid, 0], src_v.at[0], isem0)
        pltpu.async_copy(dst_h.at[wid, 0], dst_v.at[0], isem0)
        pltpu.async_copy(src_h.at[wid, 1], src_v.at[1], isem1)
        pltpu.async_copy(dst_h.at[wid, 1], dst_v.at[1], isem1)

        @pl.loop(0, NGRP, step=2)
        def _(g):
            for gb in range(2):
                sv = src_v.at[gb]
                dv = dst_v.at[gb]
                pltpu.make_async_copy(src_h.at[wid, 0], sv, isems[gb]).wait()
                pltpu.make_async_copy(dst_h.at[wid, 0], dv, isems[gb]).wait()
                dg = [
                    pltpu.async_copy(table_h.at[sv.at[0]], gbuf0, gsem0),
                    pltpu.async_copy(table_h.at[sv.at[1]], gbuf1, gsem1),
                ]
                for j in range(GRP):
                    p = j % 2
                    dg[p].wait()
                    ds = pltpu.async_copy(bufs[p], acc_sh.at[dv.at[j]],
                                          ssems[p], add=True)
                    ds.wait()
                    if j + 2 < GRP:
                        dg[p] = pltpu.async_copy(table_h.at[sv.at[j + 2]],
                                                 bufs[p], gsems[p])
                gn = g + gb + 2

                @pl.when(gn < NGRP)
                def _():
                    pltpu.async_copy(src_h.at[wid, gn], sv, isems[gb])
                    pltpu.async_copy(dst_h.at[wid, gn], dv, isems[gb])

        plsc.subcore_barrier()

        for j in range(ROWS_PER_TILE // KCH):
            r0 = base + j * KCH
            pltpu.sync_copy(acc_sh.at[pl.ds(r0, KCH)], out_h.at[cid, pl.ds(r0, KCH)])

    mesh = plsc.VectorSubcoreMesh(core_axis_name="c", subcore_axis_name="s")
    params = None
    if width % 128 != 0:
        params = pltpu.CompilerParams(use_tc_tiling_on_sc=False)
    k = pl.kernel(
        body,
        out_type=jax.ShapeDtypeStruct((NUM_SC, N_PAD, width), jnp.float32),
        mesh=mesh,
        scratch_types=[
            pltpu.VMEM((2, GRP, KCH), jnp.int32),
            pltpu.VMEM((2, GRP, KCH), jnp.int32),
            pltpu.VMEM((KCH, width), jnp.float32),
            pltpu.VMEM((KCH, width), jnp.float32),
            pltpu.VMEM_SHARED((N_PAD, width), jnp.float32),
            pltpu.SemaphoreType.DMA,
            pltpu.SemaphoreType.DMA,
            pltpu.SemaphoreType.DMA,
            pltpu.SemaphoreType.DMA,
            pltpu.SemaphoreType.DMA,
            pltpu.SemaphoreType.DMA,
        ],
        compiler_params=params)
    return k(table, srcr, dstr)


def _sc_degree(dstr):
    def body(dst_h, deg_h, dst_v, ones_v, deg_sh, ssem, isem0, isem1):
        cid = lax.axis_index("c")
        sid = lax.axis_index("s")
        wid = cid * NSUB + sid
        zv = jnp.zeros((16,), jnp.float32)

        @pl.loop(0, KCH)
        def _(r):
            ones_v[r, :] = zv

        base = sid * ROWS_PER_TILE
        for j in range(ROWS_PER_TILE // KCH):
            pltpu.sync_copy(ones_v, deg_sh.at[pl.ds(base + j * KCH, KCH)])
        ov = jnp.ones((16,), jnp.float32)

        @pl.loop(0, KCH)
        def _(r):
            ones_v[r, :] = ov

        plsc.subcore_barrier()
        isems = (isem0, isem1)
        pltpu.async_copy(dst_h.at[wid, 0], dst_v.at[0], isem0)
        pltpu.async_copy(dst_h.at[wid, 1], dst_v.at[1], isem1)

        @pl.loop(0, NGRP, step=2)
        def _(g):
            for gb in range(2):
                dv = dst_v.at[gb]
                pltpu.make_async_copy(dst_h.at[wid, 0], dv, isems[gb]).wait()
                descs = [
                    pltpu.async_copy(ones_v, deg_sh.at[dv.at[j]], ssem, add=True)
                    for j in range(GRP)
                ]
                for d in descs:
                    d.wait()
                gn = g + gb + 2

                @pl.when(gn < NGRP)
                def _():
                    pltpu.async_copy(dst_h.at[wid, gn], dv, isems[gb])

        plsc.subcore_barrier()
        for j in range(ROWS_PER_TILE // KCH):
            r0 = base + j * KCH
            pltpu.sync_copy(deg_sh.at[pl.ds(r0, KCH)], deg_h.at[cid, pl.ds(r0, KCH)])

    mesh = plsc.VectorSubcoreMesh(core_axis_name="c", subcore_axis_name="s")
    k = pl.kernel(
        body,
        out_type=jax.ShapeDtypeStruct((NUM_SC, N_PAD, 16), jnp.float32),
        mesh=mesh,
        scratch_types=[
            pltpu.VMEM((2, GRP, KCH), jnp.int32),
            pltpu.VMEM((KCH, 16), jnp.float32),
            pltpu.VMEM_SHARED((N_PAD, 16), jnp.float32),
            pltpu.SemaphoreType.DMA,
            pltpu.SemaphoreType.DMA,
            pltpu.SemaphoreType.DMA,
        ],
        compiler_params=pltpu.CompilerParams(use_tc_tiling_on_sc=False))
    return k(dstr)


def _dense(x, Ws, Wn, b):
    def body(x_ref, ws_ref, wn_ref, b_ref, hs_ref, hn_ref):
        xb = x_ref[...]
        hs_ref[...] = jnp.dot(xb, ws_ref[...], preferred_element_type=jnp.float32) + b_ref[...]
        hn_ref[...] = jnp.dot(xb, wn_ref[...], preferred_element_type=jnp.float32)

    return pl.pallas_call(
        body,
        grid=(N_PAD // BM,),
        in_specs=[
            pl.BlockSpec((BM, FEAT), lambda i: (i, 0)),
            pl.BlockSpec((FEAT, FEAT), lambda i: (0, 0)),
            pl.BlockSpec((FEAT, FEAT), lambda i: (0, 0)),
            pl.BlockSpec((1, FEAT), lambda i: (0, 0)),
        ],
        out_specs=[pl.BlockSpec((BM, FEAT), lambda i: (i, 0))] * 2,
        out_shape=[jax.ShapeDtypeStruct((N_PAD, FEAT), jnp.float32)] * 2,
    )(x, Ws, Wn, b.reshape(1, FEAT))


def _combine_dense(hs_prev, aggp, degp, Ws, Wn, b):
    def body(hs_ref, ag_ref, dg_ref, ws_ref, wn_ref, b_ref, hs2_ref, hn2_ref):
        agg = ag_ref[0] + ag_ref[1]
        deg = dg_ref[0, :, 0:1] + dg_ref[1, :, 0:1]
        invd = 1.0 / jnp.maximum(deg, 1.0)
        h = jnp.maximum(hs_ref[...] + agg * invd, 0.0)
        hs2_ref[...] = jnp.dot(h, ws_ref[...], preferred_element_type=jnp.float32) + b_ref[...]
        hn2_ref[...] = jnp.dot(h, wn_ref[...], preferred_element_type=jnp.float32)

    ws_w = Ws.shape[1]
    wn_w = Wn.shape[1]
    return pl.pallas_call(
        body,
        grid=(N_PAD // BM,),
        in_specs=[
            pl.BlockSpec((BM, FEAT), lambda i: (i, 0)),
            pl.BlockSpec((NUM_SC, BM, FEAT), lambda i: (0, i, 0)),
            pl.BlockSpec((NUM_SC, BM, 16), lambda i: (0, i, 0)),
            pl.BlockSpec((FEAT, ws_w), lambda i: (0, 0)),
            pl.BlockSpec((FEAT, wn_w), lambda i: (0, 0)),
            pl.BlockSpec((1, ws_w), lambda i: (0, 0)),
        ],
        out_specs=[
            pl.BlockSpec((BM, ws_w), lambda i: (i, 0)),
            pl.BlockSpec((BM, wn_w), lambda i: (i, 0)),
        ],
        out_shape=[
            jax.ShapeDtypeStruct((N_PAD, ws_w), jnp.float32),
            jax.ShapeDtypeStruct((N_PAD, wn_w), jnp.float32),
        ],
    )(hs_prev, aggp, degp, Ws, Wn, b.reshape(1, ws_w))


def _final_combine(hs3, aggp, degp):
    def body(hs_ref, ag_ref, dg_ref, o_ref):
        agg = ag_ref[0] + ag_ref[1]
        deg = dg_ref[0, :, 0:1] + dg_ref[1, :, 0:1]
        invd = 1.0 / jnp.maximum(deg, 1.0)
        o_ref[...] = hs_ref[...] + agg * invd

    return pl.pallas_call(
        body,
        grid=(N_PAD // BM,),
        in_specs=[
            pl.BlockSpec((BM, CP), lambda i: (i, 0)),
            pl.BlockSpec((NUM_SC, BM, CP), lambda i: (0, i, 0)),
            pl.BlockSpec((NUM_SC, BM, 16), lambda i: (0, i, 0)),
        ],
        out_specs=pl.BlockSpec((BM, CP), lambda i: (i, 0)),
        out_shape=jax.ShapeDtypeStruct((N_PAD, CP), jnp.float32),
    )(hs3, aggp, degp)


def kernel(x, edge_index, Ws1, Wn1, b1, Ws2, Wn2, b2, Ws3, Wn3, b3):
    src = edge_index[0].astype(jnp.int32)
    dst = edge_index[1].astype(jnp.int32)
    pad = E_PAD - N_EDGES
    pad_ids = jnp.arange(pad, dtype=jnp.int32)
    srcr = jnp.concatenate([src, pad_ids % N_NODES]).reshape(
        NUM_SC * NSUB, NGRP, GRP, KCH)
    dstr = jnp.concatenate([dst, N_NODES + pad_ids % (N_PAD - N_NODES)]).reshape(
        NUM_SC * NSUB, NGRP, GRP, KCH)
    xp = jnp.pad(x, ((0, N_PAD - N_NODES), (0, 0)))
    Ws3p = jnp.pad(Ws3, ((0, 0), (0, CP - N_CLS)))
    Wn3p = jnp.pad(Wn3, ((0, 0), (0, CP - N_CLS)))
    b3p = jnp.pad(b3, (0, CP - N_CLS))

    degp = _sc_degree(dstr)
    hs1, hn1 = _dense(xp, Ws1, Wn1, b1)
    agg1 = _sc_aggregate(hn1, srcr, dstr)
    hs2, hn2 = _combine_dense(hs1, agg1, degp, Ws2, Wn2, b2)
    agg2 = _sc_aggregate(hn2, srcr, dstr)
    hs3, hn3 = _combine_dense(hs2, agg2, degp, Ws3p, Wn3p, b3p)
    agg3 = _sc_aggregate(hn3, srcr, dstr, width=CP)
    out = _final_combine(hs3, agg3, degp)
    return out[:N_NODES, :N_CLS]


# speedup vs baseline: 13.0456x; 1.0005x over previous
import jax
import jax.numpy as jnp
from jax import lax
from jax.experimental import pallas as pl
from jax.experimental.pallas import tpu as pltpu
from jax.experimental.pallas import tpu_sc as plsc

N_NODES = 10000
FEAT = 128
N_CLS = 47
CP = 48
N_PAD = 10240
N_EDGES = 320000
NUM_SC = 2
NSUB = 16
KCH = 128
GRP = 8
NGRP = 10
E_PAD = NUM_SC * NSUB * NGRP * GRP * KCH
ROWS_PER_TILE = N_PAD // NSUB
BM = 1024


def _sc_aggregate(table, srcr, dstr, width=FEAT):
    def body(table_h, src_h, dst_h, out_h, src_v0, src_v1, dst_v0, dst_v1,
             gbuf0, gbuf1, acc_sh, gsem0, gsem1, ssem0, ssem1, isem0, isem1):
        cid = lax.axis_index("c")
        sid = lax.axis_index("s")
        wid = cid * NSUB + sid

        zv = jnp.zeros((16,), jnp.float32)

        @pl.loop(0, KCH)
        def _(r):
            for c2 in range(width // 16):
                gbuf0[r, pl.ds(c2 * 16, 16)] = zv

        base = sid * ROWS_PER_TILE
        for j in range(ROWS_PER_TILE // KCH):
            pltpu.sync_copy(gbuf0, acc_sh.at[pl.ds(base + j * KCH, KCH)])
        plsc.subcore_barrier()

        bufs = (gbuf0, gbuf1)
        gsems = (gsem0, gsem1)
        ssems = (ssem0, ssem1)
        isems = (isem0, isem1)

        srcs = (src_v0, src_v1)
        dsts = (dst_v0, dst_v1)
        pltpu.async_copy(src_h.at[wid, 0], src_v0, isem0)
        pltpu.async_copy(dst_h.at[wid, 0], dst_v0, isem0)
        pltpu.async_copy(src_h.at[wid, 1], src_v1, isem1)
        pltpu.async_copy(dst_h.at[wid, 1], dst_v1, isem1)

        @pl.loop(0, NGRP, step=2)
        def _(g):
            for gb in range(2):
                sv = srcs[gb]
                dv = dsts[gb]
                pltpu.make_async_copy(src_h.at[wid, 0], sv, isems[gb]).wait()
                pltpu.make_async_copy(dst_h.at[wid, 0], dv, isems[gb]).wait()
                dg = [
                    pltpu.async_copy(table_h.at[sv.at[0]], gbuf0, gsem0),
                    pltpu.async_copy(table_h.at[sv.at[1]], gbuf1, gsem1),
                ]
                for j in range(GRP):
                    p = j % 2
                    dg[p].wait()
                    ds = pltpu.async_copy(bufs[p], acc_sh.at[dv.at[j]],
                                          ssems[p], add=True)
                    ds.wait()
                    if j + 2 < GRP:
                        dg[p] = pltpu.async_copy(table_h.at[sv.at[j + 2]],
                                                 bufs[p], gsems[p])
                gn = g + gb + 2

                @pl.when(gn < NGRP)
                def _():
                    pltpu.async_copy(src_h.at[wid, gn], sv, isems[gb])
                    pltpu.async_copy(dst_h.at[wid, gn], dv, isems[gb])

        plsc.subcore_barrier()

        for j in range(ROWS_PER_TILE // KCH):
            r0 = base + j * KCH
            pltpu.sync_copy(acc_sh.at[pl.ds(r0, KCH)], out_h.at[cid, pl.ds(r0, KCH)])

    mesh = plsc.VectorSubcoreMesh(core_axis_name="c", subcore_axis_name="s")
    params = None
    if width % 128 != 0:
        params = pltpu.CompilerParams(use_tc_tiling_on_sc=False)
    k = pl.kernel(
        body,
        out_type=jax.ShapeDtypeStruct((NUM_SC, N_PAD, width), jnp.float32),
        mesh=mesh,
        scratch_types=[
            pltpu.VMEM((GRP, KCH), jnp.int32),
            pltpu.VMEM((GRP, KCH), jnp.int32),
            pltpu.VMEM((GRP, KCH), jnp.int32),
            pltpu.VMEM((GRP, KCH), jnp.int32),
            pltpu.VMEM((KCH, width), jnp.float32),
            pltpu.VMEM((KCH, width), jnp.float32),
            pltpu.VMEM_SHARED((N_PAD, width), jnp.float32),
            pltpu.SemaphoreType.DMA,
            pltpu.SemaphoreType.DMA,
            pltpu.SemaphoreType.DMA,
            pltpu.SemaphoreType.DMA,
            pltpu.SemaphoreType.DMA,
            pltpu.SemaphoreType.DMA,
        ],
        compiler_params=params)
    return k(table, srcr, dstr)


def _sc_degree(dstr):
    def body(dst_h, deg_h, dst_v0, dst_v1, ones_v, deg_sh, ssem, isem0, isem1):
        cid = lax.axis_index("c")
        sid = lax.axis_index("s")
        wid = cid * NSUB + sid
        zv = jnp.zeros((16,), jnp.float32)

        @pl.loop(0, KCH)
        def _(r):
            ones_v[r, :] = zv

        base = sid * ROWS_PER_TILE
        for j in range(ROWS_PER_TILE // KCH):
            pltpu.sync_copy(ones_v, deg_sh.at[pl.ds(base + j * KCH, KCH)])
        ov = jnp.ones((16,), jnp.float32)

        @pl.loop(0, KCH)
        def _(r):
            ones_v[r, :] = ov

        plsc.subcore_barrier()
        isems = (isem0, isem1)
        dsts = (dst_v0, dst_v1)
        pltpu.async_copy(dst_h.at[wid, 0], dst_v0, isem0)
        pltpu.async_copy(dst_h.at[wid, 1], dst_v1, isem1)

        @pl.loop(0, NGRP, step=2)
        def _(g):
            for gb in range(2):
                dv = dsts[gb]
                pltpu.make_async_copy(dst_h.at[wid, 0], dv, isems[gb]).wait()
                descs = [
                    pltpu.async_copy(ones_v, deg_sh.at[dv.at[j]], ssem, add=True)
                    for j in range(GRP)
                ]
                for d in descs:
                    d.wait()
                gn = g + gb + 2

                @pl.when(gn < NGRP)
                def _():
                    pltpu.async_copy(dst_h.at[wid, gn], dv, isems[gb])

        plsc.subcore_barrier()
        for j in range(ROWS_PER_TILE // KCH):
            r0 = base + j * KCH
            pltpu.sync_copy(deg_sh.at[pl.ds(r0, KCH)], deg_h.at[cid, pl.ds(r0, KCH)])

    mesh = plsc.VectorSubcoreMesh(core_axis_name="c", subcore_axis_name="s")
    k = pl.kernel(
        body,
        out_type=jax.ShapeDtypeStruct((NUM_SC, N_PAD, 16), jnp.float32),
        mesh=mesh,
        scratch_types=[
            pltpu.VMEM((GRP, KCH), jnp.int32),
            pltpu.VMEM((GRP, KCH), jnp.int32),
            pltpu.VMEM((KCH, 16), jnp.float32),
            pltpu.VMEM_SHARED((N_PAD, 16), jnp.float32),
            pltpu.SemaphoreType.DMA,
            pltpu.SemaphoreType.DMA,
            pltpu.SemaphoreType.DMA,
        ],
        compiler_params=pltpu.CompilerParams(use_tc_tiling_on_sc=False))
    return k(dstr)


def _dense(x, Ws, Wn, b):
    def body(x_ref, ws_ref, wn_ref, b_ref, hs_ref, hn_ref):
        xb = x_ref[...]
        hs_ref[...] = jnp.dot(xb, ws_ref[...], preferred_element_type=jnp.float32) + b_ref[...]
        hn_ref[...] = jnp.dot(xb, wn_ref[...], preferred_element_type=jnp.float32)

    return pl.pallas_call(
        body,
        grid=(N_PAD // BM,),
        in_specs=[
            pl.BlockSpec((BM, FEAT), lambda i: (i, 0)),
            pl.BlockSpec((FEAT, FEAT), lambda i: (0, 0)),
            pl.BlockSpec((FEAT, FEAT), lambda i: (0, 0)),
            pl.BlockSpec((1, FEAT), lambda i: (0, 0)),
        ],
        out_specs=[pl.BlockSpec((BM, FEAT), lambda i: (i, 0))] * 2,
        out_shape=[jax.ShapeDtypeStruct((N_PAD, FEAT), jnp.float32)] * 2,
    )(x, Ws, Wn, b.reshape(1, FEAT))


def _combine_dense(hs_prev, aggp, degp, Ws, Wn, b):
    def body(hs_ref, ag_ref, dg_ref, ws_ref, wn_ref, b_ref, hs2_ref, hn2_ref):
        agg = ag_ref[0] + ag_ref[1]
        deg = dg_ref[0, :, 0:1] + dg_ref[1, :, 0:1]
        invd = 1.0 / jnp.maximum(deg, 1.0)
        h = jnp.maximum(hs_ref[...] + agg * invd, 0.0)
        hs2_ref[...] = jnp.dot(h, ws_ref[...], preferred_element_type=jnp.float32) + b_ref[...]
        hn2_ref[...] = jnp.dot(h, wn_ref[...], preferred_element_type=jnp.float32)

    ws_w = Ws.shape[1]
    wn_w = Wn.shape[1]
    return pl.pallas_call(
        body,
        grid=(N_PAD // BM,),
        in_specs=[
            pl.BlockSpec((BM, FEAT), lambda i: (i, 0)),
            pl.BlockSpec((NUM_SC, BM, FEAT), lambda i: (0, i, 0)),
            pl.BlockSpec((NUM_SC, BM, 16), lambda i: (0, i, 0)),
            pl.BlockSpec((FEAT, ws_w), lambda i: (0, 0)),
            pl.BlockSpec((FEAT, wn_w), lambda i: (0, 0)),
            pl.BlockSpec((1, ws_w), lambda i: (0, 0)),
        ],
        out_specs=[
            pl.BlockSpec((BM, ws_w), lambda i: (i, 0)),
            pl.BlockSpec((BM, wn_w), lambda i: (i, 0)),
        ],
        out_shape=[
            jax.ShapeDtypeStruct((N_PAD, ws_w), jnp.float32),
            jax.ShapeDtypeStruct((N_PAD, wn_w), jnp.float32),
        ],
    )(hs_prev, aggp, degp, Ws, Wn, b.reshape(1, ws_w))


def _final_combine(hs3, aggp, degp):
    def body(hs_ref, ag_ref, dg_ref, o_ref):
        agg = ag_ref[0] + ag_ref[1]
        deg = dg_ref[0, :, 0:1] + dg_ref[1, :, 0:1]
        invd = 1.0 / jnp.maximum(deg, 1.0)
        o_ref[...] = hs_ref[...] + agg * invd

    return pl.pallas_call(
        body,
        grid=(N_PAD // BM,),
        in_specs=[
            pl.BlockSpec((BM, CP), lambda i: (i, 0)),
            pl.BlockSpec((NUM_SC, BM, CP), lambda i: (0, i, 0)),
            pl.BlockSpec((NUM_SC, BM, 16), lambda i: (0, i, 0)),
        ],
        out_specs=pl.BlockSpec((BM, CP), lambda i: (i, 0)),
        out_shape=jax.ShapeDtypeStruct((N_PAD, CP), jnp.float32),
    )(hs3, aggp, degp)


def kernel(x, edge_index, Ws1, Wn1, b1, Ws2, Wn2, b2, Ws3, Wn3, b3):
    src = edge_index[0].astype(jnp.int32)
    dst = edge_index[1].astype(jnp.int32)
    pad = E_PAD - N_EDGES
    pad_ids = jnp.arange(pad, dtype=jnp.int32)
    srcr = jnp.concatenate([src, pad_ids % N_NODES]).reshape(
        NUM_SC * NSUB, NGRP, GRP, KCH)
    dstr = jnp.concatenate([dst, N_NODES + pad_ids % (N_PAD - N_NODES)]).reshape(
        NUM_SC * NSUB, NGRP, GRP, KCH)
    xp = jnp.pad(x, ((0, N_PAD - N_NODES), (0, 0)))
    Ws3p = jnp.pad(Ws3, ((0, 0), (0, CP - N_CLS)))
    Wn3p = jnp.pad(Wn3, ((0, 0), (0, CP - N_CLS)))
    b3p = jnp.pad(b3, (0, CP - N_CLS))

    degp = _sc_degree(dstr)
    hs1, hn1 = _dense(xp, Ws1, Wn1, b1)
    agg1 = _sc_aggregate(hn1, srcr, dstr)
    hs2, hn2 = _combine_dense(hs1, agg1, degp, Ws2, Wn2, b2)
    agg2 = _sc_aggregate(hn2, srcr, dstr)
    hs3, hn3 = _combine_dense(hs2, agg2, degp, Ws3p, Wn3p, b3p)
    agg3 = _sc_aggregate(hn3, srcr, dstr, width=CP)
    out = _final_combine(hs3, agg3, degp)
    return out[:N_NODES, :N_CLS]
